# Initial kernel scaffold; baseline (speedup 1.0000x reference)
import jax
import jax.numpy as jnp
from jax.experimental import pallas as pl


def kernel(x, edge_index, params):
    raise NotImplementedError("write your pallas kernel here")


# speedup vs baseline: 5.5473x; 5.5473x over previous
import functools
import jax
import jax.numpy as jnp
from jax import lax
from jax.experimental import pallas as pl

_N = 2048
_RB = 256
_NB = _N // _RB


def _prep_body(c_ref, on_ref, in_ref):
    c = c_ref[...]
    deg_out = jnp.sum(c, axis=0)
    deg_in = jnp.sum(c, axis=1)
    on_ref[...] = (1.0 / jnp.sqrt(jnp.maximum(deg_out, 1.0)))[:, None]
    in_ref[...] = (1.0 / jnp.sqrt(jnp.maximum(deg_in, 1.0)))[:, None]


def _prep(C):
    return pl.pallas_call(
        _prep_body,
        out_shape=(jax.ShapeDtypeStruct((_N, 1), jnp.float32),
                   jax.ShapeDtypeStruct((_N, 1), jnp.float32)),
    )(C)


def _gconv_body(c_ref, h_ref, on_ref, inn_ref, w_ref, b_ref, o_ref):
    hs = h_ref[...] * on_ref[...]
    agg = jnp.dot(c_ref[...], hs, preferred_element_type=jnp.float32)
    agg = agg * inn_ref[...]
    o_ref[...] = jnp.dot(agg, w_ref[...], preferred_element_type=jnp.float32) + b_ref[...]


def _gconv(C, h, on, inn, W, b):
    din, dout = W.shape
    return pl.pallas_call(
        _gconv_body,
        grid=(_NB,),
        in_specs=[
            pl.BlockSpec((_RB, _N), lambda i: (i, 0)),
            pl.BlockSpec((_N, din), lambda i: (0, 0)),
            pl.BlockSpec((_N, 1), lambda i: (0, 0)),
            pl.BlockSpec((_RB, 1), lambda i: (i, 0)),
            pl.BlockSpec((din, dout), lambda i: (0, 0)),
            pl.BlockSpec((1, dout), lambda i: (0, 0)),
        ],
        out_specs=pl.BlockSpec((_RB, dout), lambda i: (i, 0)),
        out_shape=jax.ShapeDtypeStruct((_N, dout), jnp.float32),
    )(C, h, on, inn, W, b.reshape(1, -1))


def _mm_body(h_ref, w_ref, o_ref):
    o_ref[...] = jnp.dot(h_ref[...], w_ref[...], preferred_element_type=jnp.float32)


def _matmul(h, W):
    din, dout = W.shape
    if din * dout * 4 > 24 * 1024 * 1024:
        cb = 1920
        assert dout % cb == 0
    else:
        cb = dout
    ncb = dout // cb
    return pl.pallas_call(
        _mm_body,
        grid=(_NB, ncb),
        in_specs=[
            pl.BlockSpec((_RB, din), lambda i, j: (i, 0)),
            pl.BlockSpec((din, cb), lambda i, j: (0, j)),
        ],
        out_specs=pl.BlockSpec((_RB, cb), lambda i, j: (i, j)),
        out_shape=jax.ShapeDtypeStruct((_N, dout), jnp.float32),
    )(h, W)


def _gat_body(f_ref, fb_ref, al_ref, ar_ref, c_ref, b_ref, o_ref):
    f = f_ref[0]
    fb = fb_ref[0]
    al = al_ref[0]
    ar = ar_ref[0]
    el = jnp.dot(f, al.T, preferred_element_type=jnp.float32)
    er_blk = jnp.dot(fb, ar.T, preferred_element_type=jnp.float32)
    e = er_blk + el.T
    e = jnp.where(e >= 0, e, 0.2 * e)
    c = c_ref[...]
    e = jnp.where(c > 0, e, -1e30)
    emax = jnp.max(e, axis=1, keepdims=True)
    emax = jnp.where(emax > -1e29, emax, 0.0)
    w = c * jnp.exp(e - emax)
    den = jnp.sum(w, axis=1, keepdims=True)
    alpha = w / (den + 1e-9)
    o_ref[0] = jnp.dot(alpha, f, preferred_element_type=jnp.float32) + b_ref[0]


def _gat(C, h, W, al, ar, b):
    nh, d = al.shape
    f = _matmul(h, W)
    fT = f.reshape(_N, nh, d).transpose(1, 0, 2)
    o = pl.pallas_call(
        _gat_body,
        grid=(nh, _NB),
        in_specs=[
            pl.BlockSpec((1, _N, d), lambda hh, i: (hh, 0, 0)),
            pl.BlockSpec((1, _RB, d), lambda hh, i: (hh, i, 0)),
            pl.BlockSpec((1, 1, d), lambda hh, i: (hh, 0, 0)),
            pl.BlockSpec((1, 1, d), lambda hh, i: (hh, 0, 0)),
            pl.BlockSpec((_RB, _N), lambda hh, i: (i, 0)),
            pl.BlockSpec((1, 1, d), lambda hh, i: (hh, 0, 0)),
        ],
        out_specs=pl.BlockSpec((1, _RB, d), lambda hh, i: (hh, i, 0)),
        out_shape=jax.ShapeDtypeStruct((nh, _N, d), jnp.float32),
    )(fT, fT, al.reshape(nh, 1, d), ar.reshape(nh, 1, d), C, b.reshape(nh, 1, d))
    return o.transpose(1, 0, 2).reshape(_N, nh * d)


_CB = 64


def _enc_body(x_ref, w1_ref, b1_ref, w2_ref, b2_ref, o_ref):
    x = x_ref[...]
    w1 = w1_ref[...]
    acc = jnp.zeros((_CB * 64, 64), jnp.float32)
    for di in range(3):
        for dj in range(3):
            p = x[:, di:di + 8, dj:dj + 8, :].reshape(_CB * 64, 96)
            acc = acc + jnp.dot(p, w1[di, dj], preferred_element_type=jnp.float32)
    y = jnp.maximum(acc + b1_ref[...], 0.0).reshape(_CB, 8, 8, 64)
    y = y.reshape(_CB, 4, 2, 4, 2, 64).max(axis=4).max(axis=2)
    w2 = w2_ref[...]
    acc2 = jnp.zeros((_CB * 4, 32), jnp.float32)
    for di in range(3):
        for dj in range(3):
            p = y[:, di:di + 2, dj:dj + 2, :].reshape(_CB * 4, 64)
            acc2 = acc2 + jnp.dot(p, w2[di, dj], preferred_element_type=jnp.float32)
    z = jnp.maximum(acc2 + b2_ref[...], 0.0).reshape(_CB, 2, 2, 32)
    o_ref[...] = z.max(axis=2).max(axis=1)


def _encoder(h, Wc1, bc1, Wc2, bc2):
    x = h.reshape(_N, 96, 10, 10).transpose(0, 2, 3, 1)
    w1 = Wc1.transpose(2, 3, 1, 0)
    w2 = Wc2.transpose(2, 3, 1, 0)
    return pl.pallas_call(
        _enc_body,
        grid=(_N // _CB,),
        in_specs=[
            pl.BlockSpec((_CB, 10, 10, 96), lambda i: (i, 0, 0, 0)),
            pl.BlockSpec((3, 3, 96, 64), lambda i: (0, 0, 0, 0)),
            pl.BlockSpec((1, 64), lambda i: (0, 0)),
            pl.BlockSpec((3, 3, 64, 32), lambda i: (0, 0, 0, 0)),
            pl.BlockSpec((1, 32), lambda i: (0, 0)),
        ],
        out_specs=pl.BlockSpec((_CB, 32), lambda i: (i, 0)),
        out_shape=jax.ShapeDtypeStruct((_N, 32), jnp.float32),
    )(x, w1, bc1.reshape(1, -1), w2, bc2.reshape(1, -1))


def _conv_nhwc(x, w, out_hw, cin, cout, rb):
    acc = jnp.zeros((rb * out_hw * out_hw, cout), jnp.float32)
    for di in range(3):
        for dj in range(3):
            p = x[:, di:di + out_hw, dj:dj + out_hw, :].reshape(rb * out_hw * out_hw, cin)
            acc = acc + jnp.dot(p, w[di, dj], preferred_element_type=jnp.float32)
    return acc.reshape(rb, out_hw, out_hw, cout)


def _dconv_body(x_ref, w_ref, b_ref, o_ref, *, out_hw, cin, cout, act, rb):
    y = _conv_nhwc(x_ref[...], w_ref[...], out_hw, cin, cout, rb)
    y = y + b_ref[...].reshape(1, 1, 1, cout)
    if act == 'relu':
        y = jnp.maximum(y, 0.0)
    else:
        y = jnp.tanh(y)
    o_ref[...] = y


def _dconv(x, w, b, out_hw, cin, cout, act, rb):
    in_hw = out_hw + 2
    return pl.pallas_call(
        functools.partial(_dconv_body, out_hw=out_hw, cin=cin, cout=cout,
                          act=act, rb=rb),
        grid=(_N // rb,),
        in_specs=[
            pl.BlockSpec((rb, in_hw, in_hw, cin), lambda i: (i, 0, 0, 0)),
            pl.BlockSpec((3, 3, cin, cout), lambda i: (0, 0, 0, 0)),
            pl.BlockSpec((1, cout), lambda i: (0, 0)),
        ],
        out_specs=pl.BlockSpec((rb, out_hw, out_hw, cout), lambda i: (i, 0, 0, 0)),
        out_shape=jax.ShapeDtypeStruct((_N, out_hw, out_hw, cout), jnp.float32),
    )(x, w, b.reshape(1, -1))


def _decoder(h, Wd1, bd1, Wd2, bd2, Wd3, bd3, Wfc, bfc):
    x = h.reshape(_N, 48, 10, 10).transpose(0, 2, 3, 1)
    x = jnp.pad(x, ((0, 0), (2, 2), (2, 2), (0, 0)))
    w1 = jnp.flip(Wd1, (2, 3)).transpose(2, 3, 0, 1)
    w2 = jnp.flip(Wd2, (2, 3)).transpose(2, 3, 0, 1)
    w3 = jnp.flip(Wd3, (2, 3)).transpose(2, 3, 0, 1)
    y = _dconv(x, w1, bd1, 12, 48, 24, 'relu', 64)
    y = jnp.pad(y, ((0, 0), (2, 2), (2, 2), (0, 0)))
    y = _dconv(y, w2, bd2, 14, 24, 12, 'relu', 16)
    y = jnp.pad(y, ((0, 0), (2, 2), (2, 2), (0, 0)))
    y = _dconv(y, w3, bd3, 16, 12, 3, 'tanh', 16)
    wfc = Wfc.reshape(3, 16, 16, 36).transpose(1, 2, 0, 3).reshape(768, 36)
    return _matmul(y.reshape(_N, 768), wfc) + bfc


@jax.jit
def _forward(x, edge_index, p):
    src, dst = edge_index[0], edge_index[1]
    C = jnp.zeros((_N, _N), jnp.float32).at[dst, src].add(1.0)
    on, inn = _prep(C)

    h = _gconv(C, x, on, inn, p['W1'], p['b1'])
    h = _gat(C, h, p['Wg1'], p['al1'], p['ar1'], p['bg1'])
    h = _gconv(C, h, on, inn, p['W2'], p['b2'])
    h = _gat(C, h, p['Wg2'], p['al2'], p['ar2'], p['bg2'])
    h = _gconv(C, h, on, inn, p['W3'], p['b3'])
    h = _gat(C, h, p['Wg3'], p['al3'], p['ar3'], p['bg3'])
    h = _encoder(h, p['Wc1'], p['bc1'], p['Wc2'], p['bc2'])
    h = _gat(C, h, p['Wgm'], p['alm'], p['arm'], p['bgm'])
    h = _gconv(C, h, on, inn, p['W4'], p['b4'])
    h = _gat(C, h, p['Wg4'], p['al4'], p['ar4'], p['bg4'])
    h = _gconv(C, h, on, inn, p['W5'], p['b5'])
    h = _gat(C, h, p['Wg5'], p['al5'], p['ar5'], p['bg5'])
    h = _gconv(C, h, on, inn, p['W6'], p['b6'])
    h = _gat(C, h, p['Wg6'], p['al6'], p['ar6'], p['bg6'])
    return _decoder(h, p['Wd1'], p['bd1'], p['Wd2'], p['bd2'],
                    p['Wd3'], p['bd3'], p['Wfc'], p['bfc'])


def kernel(x, edge_index, params):
    return _forward(x, edge_index, params)
